# baseline (speedup 1.0000x reference)
import jax
import jax.numpy as jnp
from jax.experimental import pallas as pl
from jax.experimental.pallas import tpu as pltpu

_N = 1024
_H = _N // 2


def _simt(xh, xnh, yh):
    num = jax.lax.dot_general(
        xh, yh,
        dimension_numbers=(((1,), (1,)), ((), ())),
        preferred_element_type=jnp.float32,
    )
    yn = jnp.sqrt(jnp.sum(yh * yh, axis=1))
    denom = jnp.maximum(xnh * yn[None, :], 1e-8)
    return num / denom


def _diag_count(simt):
    row = jax.lax.broadcasted_iota(jnp.int32, (_H, _H), 0)
    col = jax.lax.broadcasted_iota(jnp.int32, (_H, _H), 1)
    d = jnp.sum(jnp.where(row == col, simt, 0.0), axis=0, keepdims=True)
    beats = (simt > d) | ((simt == d) & (row < col))
    return jnp.sum(jnp.where(beats, 1.0, 0.0), axis=0, keepdims=True), d


def _cnt(pred):
    return jnp.sum(jnp.where(pred, 1.0, 0.0), axis=0, keepdims=True)


def _accs(cnt):
    t1 = jnp.sum(jnp.where(cnt == 0.0, 1.0, 0.0), axis=1, keepdims=True)
    t10 = jnp.sum(jnp.where(cnt < 10.0, 1.0, 0.0), axis=1, keepdims=True)
    return t1, t10


def _acc_kernel(z_hbm, y_hbm, out_ref, xv, yv, sx0, sx1, sy0, sy1):
    lo = pl.ds(0, _H)
    hi = pl.ds(_H, _H)
    cx0 = pltpu.make_async_copy(z_hbm.at[lo, :], xv.at[lo, :], sx0)
    cx0.start()
    cy0 = pltpu.make_async_copy(y_hbm.at[lo, :], yv.at[lo, :], sy0)
    cy0.start()
    cx1 = pltpu.make_async_copy(z_hbm.at[hi, :], xv.at[hi, :], sx1)
    cx1.start()
    cy1 = pltpu.make_async_copy(y_hbm.at[hi, :], yv.at[hi, :], sy1)
    cy1.start()

    cx0.wait()
    x0 = xv[lo, :]
    xn0 = jnp.sqrt(jnp.sum(x0 * x0, axis=1))[:, None]

    cy0.wait()
    y0 = yv[lo, :]
    s00 = _simt(x0, xn0, y0)

    cx1.wait()
    x1 = xv[hi, :]
    xn1 = jnp.sqrt(jnp.sum(x1 * x1, axis=1))[:, None]
    s10 = _simt(x1, xn1, y0)

    cy1.wait()
    y1 = yv[hi, :]
    s11 = _simt(x1, xn1, y1)
    s01 = _simt(x0, xn0, y1)

    c00, d0 = _diag_count(s00)
    cnt0 = c00 + _cnt(s10 > d0)
    t1a, t10a = _accs(cnt0)

    c11, d1 = _diag_count(s11)
    cnt1 = c11 + _cnt(s01 >= d1)
    t1b, t10b = _accs(cnt1)

    out_ref[...] = jnp.concatenate(
        [t1a + t1b, t10a + t10b], axis=1
    ) * (1.0 / _N)


def kernel(Z, Y):
    out = pl.pallas_call(
        _acc_kernel,
        in_specs=[
            pl.BlockSpec(memory_space=pltpu.MemorySpace.HBM),
            pl.BlockSpec(memory_space=pltpu.MemorySpace.HBM),
        ],
        out_specs=pl.BlockSpec(memory_space=pltpu.MemorySpace.VMEM),
        out_shape=jax.ShapeDtypeStruct((1, 2), jnp.float32),
        scratch_shapes=[
            pltpu.VMEM((_N, _N), jnp.float32),
            pltpu.VMEM((_N, _N), jnp.float32),
            pltpu.SemaphoreType.DMA,
            pltpu.SemaphoreType.DMA,
            pltpu.SemaphoreType.DMA,
            pltpu.SemaphoreType.DMA,
        ],
    )(Z, Y)
    return (out[0, 0], out[0, 1])
